# Initial kernel scaffold; baseline (speedup 1.0000x reference)
import jax
import jax.numpy as jnp
from jax.experimental import pallas as pl


def kernel(x, table, W1, b1, W2, b2):
    raise NotImplementedError("write your pallas kernel here")


# speedup vs baseline: 2.5098x; 2.5098x over previous
import functools

import jax
import jax.numpy as jnp
from jax import lax
from jax.experimental import pallas as pl
from jax.experimental.pallas import tpu as pltpu
from jax.experimental.pallas import tpu_sc as plsc

VOCAB = 1000000
EMBED = 64
HIDDEN = 64
CLASSES = 10
BATCH = 16384
HIST = 50

NC = 2
NS = 16
LANES = 16
NW = NC * NS
B_PER_W = BATCH // NW
CB = 2
CHUNK_IDX = CB * HIST
NCHUNK = B_PER_W // CB
EV = EMBED // LANES


def _pool_body(x_hbm, table_hbm, out_hbm, idx_v, buf0, buf1, out_v, sem0, sem1):
    wid = lax.axis_index("s") * NC + lax.axis_index("c")
    base = wid * B_PER_W

    pltpu.sync_copy(x_hbm.at[wid], idx_v)

    def start(chunk, buf, sem):
        return pltpu.async_copy(table_hbm.at[idx_v.at[chunk]], buf, sem)

    def reduce_chunk(chunk, buf):
        for r in range(CB):
            row0 = r * HIST

            def body(j, accs):
                return tuple(
                    accs[k] + buf[row0 + j, pl.ds(k * LANES, LANES)]
                    for k in range(EV)
                )

            accs = lax.fori_loop(
                0, HIST, body,
                tuple(jnp.zeros((LANES,), jnp.float32) for _ in range(EV)),
            )
            orow = chunk * CB + r
            for k in range(EV):
                out_v[orow, pl.ds(k * LANES, LANES)] = accs[k] * (1.0 / HIST)

    start(0, buf0, sem0)
    start(1, buf1, sem1)

    def loop(i, carry):
        c0 = i * 2
        pltpu.make_async_copy(table_hbm.at[idx_v.at[c0]], buf0, sem0).wait()
        reduce_chunk(c0, buf0)

        @pl.when(c0 + 2 < NCHUNK)
        def _():
            start(c0 + 2, buf0, sem0)

        pltpu.make_async_copy(table_hbm.at[idx_v.at[c0 + 1]], buf1, sem1).wait()
        reduce_chunk(c0 + 1, buf1)

        @pl.when(c0 + 3 < NCHUNK)
        def _():
            start(c0 + 3, buf1, sem1)

        return carry

    lax.fori_loop(0, NCHUNK // 2, loop, 0)

    pltpu.sync_copy(out_v, out_hbm.at[pl.ds(base, B_PER_W)])


@functools.partial(jax.jit, static_argnames=())
def _sc_pool(x_grp, table):
    mesh = plsc.VectorSubcoreMesh(core_axis_name="c", subcore_axis_name="s")
    return pl.kernel(
        _pool_body,
        out_type=jax.ShapeDtypeStruct((BATCH, EMBED), jnp.float32),
        mesh=mesh,
        scratch_types=[
            pltpu.VMEM((NCHUNK, CHUNK_IDX), jnp.int32),
            pltpu.VMEM((CHUNK_IDX, EMBED), jnp.float32),
            pltpu.VMEM((CHUNK_IDX, EMBED), jnp.float32),
            pltpu.VMEM((B_PER_W, EMBED), jnp.float32),
            pltpu.SemaphoreType.DMA,
            pltpu.SemaphoreType.DMA,
        ],
        compiler_params=pltpu.CompilerParams(use_tc_tiling_on_sc=False),
    )(x_grp, table)


def _mlp_body(p_ref, w1_ref, b1_ref, w2_ref, b2_ref, o_ref):
    h = jnp.dot(p_ref[...], w1_ref[...], preferred_element_type=jnp.float32)
    h = jnp.maximum(h + b1_ref[...], 0.0)
    o_ref[...] = (
        jnp.dot(h, w2_ref[...], preferred_element_type=jnp.float32)
        + b2_ref[...]
    )


def _tc_mlp(pooled, W1, b1, W2, b2):
    blk = 2048
    grid = BATCH // blk
    return pl.pallas_call(
        _mlp_body,
        grid=(grid,),
        in_specs=[
            pl.BlockSpec((blk, EMBED), lambda i: (i, 0)),
            pl.BlockSpec((EMBED, HIDDEN), lambda i: (0, 0)),
            pl.BlockSpec((1, HIDDEN), lambda i: (0, 0)),
            pl.BlockSpec((HIDDEN, CLASSES), lambda i: (0, 0)),
            pl.BlockSpec((1, CLASSES), lambda i: (0, 0)),
        ],
        out_specs=pl.BlockSpec((blk, CLASSES), lambda i: (i, 0)),
        out_shape=jax.ShapeDtypeStruct((BATCH, CLASSES), jnp.float32),
    )(pooled, W1, b1, W2, b2)


def kernel(x, table, W1, b1, W2, b2):
    x_grp = x.astype(jnp.int32).reshape(NW, NCHUNK, CHUNK_IDX)
    pooled = _sc_pool(x_grp, table)
    return _tc_mlp(pooled, W1.reshape(EMBED, HIDDEN), b1.reshape(1, HIDDEN),
                   W2.reshape(HIDDEN, CLASSES), b2.reshape(1, CLASSES))
